# Initial kernel scaffold; baseline (speedup 1.0000x reference)
import jax
import jax.numpy as jnp
from jax.experimental import pallas as pl


def kernel(coords, params, atomids, edge_index, batch):
    raise NotImplementedError("write your pallas kernel here")


# speedup vs baseline: 1.4671x; 1.4671x over previous
import functools

import jax
import jax.numpy as jnp
from jax import lax
from jax.experimental import pallas as pl
from jax.experimental.pallas import tpu as pltpu
from jax.experimental.pallas import tpu_sc as plsc

N_NODES = 10000
N_EDGES = 320000
N_GRAPHS = 256
EMB = 128
M_DIM = 32
FF = 32
N_KERNELS = 5
MLP_DIM = 256
EDGE_IN = 2 * FF + 1 + 2 * EMB
H_DIM = 2 * EDGE_IN

NC, NS, LANES = 2, 16, 16
NW = NC * NS

E_PAD = 327680
N_PAD = 10240
BE = 2048
BN = 2000


def _sc_gather(table, idx2d):
    D = table.shape[1]
    R = idx2d.shape[0] * 128
    rows_w = R // NW
    chunk = 1024
    half = 512
    n_chunks = rows_w // chunk
    mesh = plsc.VectorSubcoreMesh(core_axis_name="c", subcore_axis_name="s")

    @functools.partial(
        pl.kernel,
        out_type=jax.ShapeDtypeStruct((R, D), table.dtype),
        mesh=mesh,
        compiler_params=pltpu.CompilerParams(use_tc_tiling_on_sc=False),
        scratch_types=[
            pltpu.VMEM((8, 128), jnp.int32),
            pltpu.VMEM((half, D), table.dtype),
            pltpu.SemaphoreType.DMA,
        ],
    )
    def k(table_hbm, idx_hbm, out_hbm, idx_v, rows_v, sem):
        wid = lax.axis_index("s") * NC + lax.axis_index("c")
        base = wid * rows_w

        def body(j, carry):
            off = pl.multiple_of(base + j * chunk, chunk)
            pltpu.sync_copy(idx_hbm.at[pl.ds(pl.multiple_of(off // 128, 8), 8)],
                            idx_v)
            for hh in range(2):
                copies = []
                for jj in range(4):
                    copies.append(pltpu.async_copy(
                        table_hbm.at[idx_v.at[hh * 4 + jj]],
                        rows_v.at[pl.ds(jj * 128, 128)], sem))
                for c in copies:
                    c.wait()
                pltpu.sync_copy(rows_v, out_hbm.at[pl.ds(off + hh * half, half)])
            return carry

        lax.fori_loop(0, n_chunks, body, 0)

    return k(table, idx2d)


def _sc_edge_dist(cx, cy, cz, src_p, dst_p):
    rows_w = E_PAD // NW
    chunk = 1024
    n_chunks = rows_w // chunk
    npad = cx.shape[0]
    mesh = plsc.VectorSubcoreMesh(core_axis_name="c", subcore_axis_name="s")

    @functools.partial(
        pl.kernel,
        out_type=jax.ShapeDtypeStruct((E_PAD,), jnp.float32),
        mesh=mesh,
        compiler_params=pltpu.CompilerParams(needs_layout_passes=False),
        scratch_types=[
            pltpu.VMEM((npad,), jnp.float32),
            pltpu.VMEM((npad,), jnp.float32),
            pltpu.VMEM((npad,), jnp.float32),
            pltpu.VMEM((chunk,), jnp.int32),
            pltpu.VMEM((chunk,), jnp.int32),
            pltpu.VMEM((chunk,), jnp.float32),
        ],
    )
    def k(cx_hbm, cy_hbm, cz_hbm, src_hbm, dst_hbm, out_hbm,
          cx_v, cy_v, cz_v, s_v, d_v, o_v):
        wid = lax.axis_index("s") * NC + lax.axis_index("c")
        base = wid * rows_w
        pltpu.sync_copy(cx_hbm, cx_v)
        pltpu.sync_copy(cy_hbm, cy_v)
        pltpu.sync_copy(cz_hbm, cz_v)

        def body(j, carry):
            off = pl.multiple_of(base + j * chunk, chunk)
            pltpu.sync_copy(src_hbm.at[pl.ds(off, chunk)], s_v)
            pltpu.sync_copy(dst_hbm.at[pl.ds(off, chunk)], d_v)

            def inner(i, c2):
                sl = pl.ds(i * LANES, LANES)
                si = s_v[sl]
                di = d_v[sl]
                rx = plsc.load_gather(cx_v, [si]) - plsc.load_gather(cx_v, [di])
                ry = plsc.load_gather(cy_v, [si]) - plsc.load_gather(cy_v, [di])
                rz = plsc.load_gather(cz_v, [si]) - plsc.load_gather(cz_v, [di])
                o_v[sl] = rx * rx + ry * ry + rz * rz
                return c2

            lax.fori_loop(0, chunk // LANES, inner, 0)
            pltpu.sync_copy(o_v, out_hbm.at[pl.ds(off, chunk)])
            return carry

        lax.fori_loop(0, n_chunks, body, 0)

    return k(cx, cy, cz, src_p, dst_p)


def _sc_scatter_add(vals, idx2d, width):
    rows_w = E_PAD // NW
    chunk = 1024
    k_sub = chunk // 128
    n_chunks = rows_w // chunk
    rpt = N_PAD // NS
    mesh = plsc.VectorSubcoreMesh(core_axis_name="c", subcore_axis_name="s")
    zeros = jnp.zeros((N_PAD, width), jnp.float32)

    @functools.partial(
        pl.kernel,
        out_type=jax.ShapeDtypeStruct((NC, N_PAD, width), jnp.float32),
        mesh=mesh,
        compiler_params=pltpu.CompilerParams(use_tc_tiling_on_sc=False),
        scratch_types=[
            pltpu.VMEM((k_sub, 128), jnp.int32),
            pltpu.VMEM((chunk, width), jnp.float32),
            pltpu.VMEM_SHARED((N_PAD, width), jnp.float32),
        ],
    )
    def k(vals_hbm, idx_hbm, zeros_hbm, out_hbm, idx_v, vals_v, acc_sh):
        cid = lax.axis_index("c")
        sid = lax.axis_index("s")
        wid = sid * NC + cid
        base = wid * rows_w
        row0 = pl.multiple_of(sid * rpt, rpt)
        pltpu.sync_copy(zeros_hbm.at[pl.ds(row0, rpt)],
                        acc_sh.at[pl.ds(row0, rpt)])
        plsc.subcore_barrier()

        def body(j, carry):
            off = pl.multiple_of(base + j * chunk, chunk)
            pltpu.sync_copy(
                idx_hbm.at[pl.ds(pl.multiple_of(off // 128, k_sub), k_sub)],
                idx_v)
            pltpu.sync_copy(vals_hbm.at[pl.ds(off, chunk)], vals_v)
            for jj in range(k_sub):
                pltpu.sync_copy(vals_v.at[pl.ds(jj * 128, 128)],
                                acc_sh.at[idx_v.at[jj]], add=True)
            return carry

        lax.fori_loop(0, n_chunks, body, 0)
        plsc.subcore_barrier()
        pltpu.sync_copy(acc_sh.at[pl.ds(row0, rpt)],
                        out_hbm.at[cid, pl.ds(row0, rpt)])

    return k(vals, idx2d, zeros)


HI = lax.Precision.HIGHEST


def _ln(x, g, b):
    mu = jnp.mean(x, axis=-1, keepdims=True)
    var = jnp.mean((x - mu) ** 2, axis=-1, keepdims=True)
    return (x - mu) * lax.rsqrt(var + 1e-5) * g + b


def _emb_body(aid_ref, emb_ref, o_ref):
    aid = aid_ref[...]
    onehot = (aid == lax.broadcasted_iota(jnp.int32, (1, 16), 1).astype(jnp.float32))
    o_ref[...] = jnp.dot(onehot.astype(jnp.float32), emb_ref[...],
                         preferred_element_type=jnp.float32, precision=HI)


def _embed(atomids_f, emb_pad):
    return pl.pallas_call(
        _emb_body,
        grid=(N_NODES // BN,),
        in_specs=[
            pl.BlockSpec((BN, 1), lambda i: (i, 0)),
            pl.BlockSpec((16, EMB), lambda i: (0, 0)),
        ],
        out_specs=pl.BlockSpec((BN, EMB), lambda i: (i, 0)),
        out_shape=jax.ShapeDtypeStruct((N_NODES, EMB), jnp.float32),
    )(atomids_f, emb_pad)


def _ea_body(rd_ref, scales_ref, o_ref):
    rd = rd_ref[...]
    x = rd / scales_ref[...]
    enc = jnp.concatenate(
        [jnp.sin(x), jnp.cos(x), rd,
         jnp.zeros((x.shape[0], 128 - 2 * FF - 1), jnp.float32)], axis=-1)
    o_ref[...] = enc.astype(jnp.bfloat16)


def _edge_attr(rd, scales):
    nblk = E_PAD // BE
    return pl.pallas_call(
        _ea_body,
        grid=(nblk,),
        in_specs=[
            pl.BlockSpec((BE, 1), lambda i: (i, 0)),
            pl.BlockSpec((1, FF), lambda i: (0, 0)),
        ],
        out_specs=pl.BlockSpec((BE, 128), lambda i: (i, 0)),
        out_shape=jax.ShapeDtypeStruct((E_PAD, 128), jnp.bfloat16),
    )(rd, scales)


def _edge_body(width, xj_ref, xi_ref, ea_ref, wi_ref, wj_ref, wea_ref,
               be1_ref, we2_ref, be2_ref, g_ref, b_ref, o_ref):
    xi = xi_ref[...].astype(jnp.bfloat16)
    xj = xj_ref[...].astype(jnp.bfloat16)
    acc = (jnp.dot(xi, wi_ref[...], preferred_element_type=jnp.float32)
           + jnp.dot(xj, wj_ref[...], preferred_element_type=jnp.float32)
           + jnp.dot(ea_ref[...], wea_ref[...], preferred_element_type=jnp.float32)
           + be1_ref[...])
    h = jax.nn.silu(acc).astype(jnp.bfloat16)
    m = jax.nn.silu(jnp.dot(h, we2_ref[...], preferred_element_type=jnp.float32)
                    + be2_ref[...])
    mn = _ln(m, g_ref[...], b_ref[...])
    if width > M_DIM:
        mn = jnp.concatenate(
            [mn, jnp.ones((mn.shape[0], 1), jnp.float32),
             jnp.zeros((mn.shape[0], width - M_DIM - 1), jnp.float32)], axis=-1)
    o_ref[...] = mn


def _edge_mlp(gathered, ea, wi, wj, wea, be1, we2, be2, g, b, width):
    nblk = E_PAD // BE
    return pl.pallas_call(
        functools.partial(_edge_body, width),
        grid=(nblk,),
        in_specs=[
            pl.BlockSpec((BE, EMB), lambda i: (i, 0)),
            pl.BlockSpec((BE, EMB), lambda i, n=nblk: (i + n, 0)),
            pl.BlockSpec((BE, 128), lambda i: (i, 0)),
            pl.BlockSpec((EMB, H_DIM), lambda i: (0, 0)),
            pl.BlockSpec((EMB, H_DIM), lambda i: (0, 0)),
            pl.BlockSpec((128, H_DIM), lambda i: (0, 0)),
            pl.BlockSpec((1, H_DIM), lambda i: (0, 0)),
            pl.BlockSpec((H_DIM, M_DIM), lambda i: (0, 0)),
            pl.BlockSpec((1, M_DIM), lambda i: (0, 0)),
            pl.BlockSpec((1, M_DIM), lambda i: (0, 0)),
            pl.BlockSpec((1, M_DIM), lambda i: (0, 0)),
        ],
        out_specs=pl.BlockSpec((BE, width), lambda i: (i, 0)),
        out_shape=jax.ShapeDtypeStruct((E_PAD, width), jnp.float32),
    )(gathered, gathered, ea, wi, wj, wea, be1, we2, be2, g, b)


def _node_body(first, s0_ref, s1_ref, cnt_ref, f_ref, whf_ref, wm_ref, bn1_ref,
               wn2_ref, bn2_ref, ge_ref, be_ref, g1_ref, b1_ref, g2_ref, b2_ref,
               o_ref, cnt_o_ref):
    s = s0_ref[0] + s1_ref[0]
    if first:
        cnt = s[:, M_DIM:M_DIM + 1]
        cnt_o_ref[...] = cnt
    else:
        cnt = cnt_ref[...]
    m_i = s[:, :M_DIM] / jnp.maximum(cnt, 1.0)
    m_i = _ln(m_i, ge_ref[...], be_ref[...])
    feats = f_ref[...]
    hf = _ln(feats, g1_ref[...], b1_ref[...])
    t = jax.nn.silu(
        jnp.dot(hf.astype(jnp.bfloat16), whf_ref[...],
                preferred_element_type=jnp.float32)
        + jnp.dot(m_i.astype(jnp.bfloat16), wm_ref[...],
                  preferred_element_type=jnp.float32)
        + bn1_ref[...])
    h2 = jnp.dot(t.astype(jnp.bfloat16), wn2_ref[...],
                 preferred_element_type=jnp.float32) + bn2_ref[...]
    h2 = _ln(h2, g2_ref[...], b2_ref[...])
    o_ref[...] = feats + h2


def _node_mlp(sums, cnt, feats, whf, wm, bn1, wn2, bn2, ge, be, g1, b1, g2, b2,
              first):
    width = sums.shape[2]
    out_shapes = [jax.ShapeDtypeStruct((N_NODES, EMB), jnp.float32),
                  jax.ShapeDtypeStruct((N_NODES, 1), jnp.float32)]
    outs = pl.pallas_call(
        functools.partial(_node_body, first),
        grid=(N_NODES // BN,),
        in_specs=[
            pl.BlockSpec((1, BN, width), lambda i: (0, i, 0)),
            pl.BlockSpec((1, BN, width), lambda i: (1, i, 0)),
            pl.BlockSpec((BN, 1), lambda i: (i, 0)),
            pl.BlockSpec((BN, EMB), lambda i: (i, 0)),
            pl.BlockSpec((EMB, 2 * EMB), lambda i: (0, 0)),
            pl.BlockSpec((M_DIM, 2 * EMB), lambda i: (0, 0)),
            pl.BlockSpec((1, 2 * EMB), lambda i: (0, 0)),
            pl.BlockSpec((2 * EMB, EMB), lambda i: (0, 0)),
            pl.BlockSpec((1, EMB), lambda i: (0, 0)),
            pl.BlockSpec((1, M_DIM), lambda i: (0, 0)),
            pl.BlockSpec((1, M_DIM), lambda i: (0, 0)),
            pl.BlockSpec((1, EMB), lambda i: (0, 0)),
            pl.BlockSpec((1, EMB), lambda i: (0, 0)),
            pl.BlockSpec((1, EMB), lambda i: (0, 0)),
            pl.BlockSpec((1, EMB), lambda i: (0, 0)),
        ],
        out_specs=[pl.BlockSpec((BN, EMB), lambda i: (i, 0)),
                   pl.BlockSpec((BN, 1), lambda i: (i, 0))],
        out_shape=out_shapes,
    )(sums, sums, cnt, feats, whf, wm, bn1, wn2, bn2, ge, be, g1, b1, g2, b2)
    return outs[0], outs[1]


def _readout_body(f0, f1, f2, f3, f4, f5, w0_ref, b0_ref, w1_ref, b1_ref,
                  w2_ref, b2_ref, bat_ref, gs_ref, gc_ref):
    flist = [f0, f1, f2, f3, f4, f5]
    acc = b0_ref[...]
    for kk in range(6):
        acc = acc + jnp.dot(flist[kk][...].astype(jnp.bfloat16),
                            w0_ref[kk * EMB:(kk + 1) * EMB, :],
                            preferred_element_type=jnp.float32)
    h = jax.nn.silu(acc)
    h = jax.nn.silu(jnp.dot(h.astype(jnp.bfloat16), w1_ref[...],
                            preferred_element_type=jnp.float32) + b1_ref[...])
    h = jax.nn.silu(jnp.dot(h.astype(jnp.bfloat16), w2_ref[...],
                            preferred_element_type=jnp.float32) + b2_ref[...])
    onehot = (bat_ref[...] == lax.broadcasted_iota(
        jnp.int32, (1, N_GRAPHS), 1).astype(jnp.float32))
    onehot = onehot.astype(jnp.float32)
    gs_part = lax.dot_general(onehot, h, (((0,), (0,)), ((), ())),
                              preferred_element_type=jnp.float32, precision=HI)
    gc_part = lax.dot_general(onehot, jnp.ones((onehot.shape[0], 1), jnp.float32),
                              (((0,), (0,)), ((), ())),
                              preferred_element_type=jnp.float32, precision=HI)

    @pl.when(pl.program_id(0) == 0)
    def _():
        gs_ref[...] = jnp.zeros_like(gs_ref)
        gc_ref[...] = jnp.zeros_like(gc_ref)

    gs_ref[...] += gs_part
    gc_ref[...] += gc_part


def _readout(flist, w0, b0, w1, b1, w2, b2, batch_f):
    spec_f = pl.BlockSpec((BN, EMB), lambda i: (i, 0))
    return pl.pallas_call(
        _readout_body,
        grid=(N_NODES // BN,),
        in_specs=[spec_f] * 6 + [
            pl.BlockSpec((6 * EMB, MLP_DIM), lambda i: (0, 0)),
            pl.BlockSpec((1, MLP_DIM), lambda i: (0, 0)),
            pl.BlockSpec((MLP_DIM, MLP_DIM), lambda i: (0, 0)),
            pl.BlockSpec((1, MLP_DIM), lambda i: (0, 0)),
            pl.BlockSpec((MLP_DIM, MLP_DIM), lambda i: (0, 0)),
            pl.BlockSpec((1, MLP_DIM), lambda i: (0, 0)),
            pl.BlockSpec((BN, 1), lambda i: (i, 0)),
        ],
        out_specs=[pl.BlockSpec((N_GRAPHS, MLP_DIM), lambda i: (0, 0)),
                   pl.BlockSpec((N_GRAPHS, 1), lambda i: (0, 0))],
        out_shape=[jax.ShapeDtypeStruct((N_GRAPHS, MLP_DIM), jnp.float32),
                   jax.ShapeDtypeStruct((N_GRAPHS, 1), jnp.float32)],
    )(*flist, w0, b0, w1, b1, w2, b2, batch_f)


def _graph_body(gs_ref, gc_ref, w0_ref, b0_ref, w1_ref, b1_ref, w2_ref, b2_ref,
                o_ref):
    g = gs_ref[...] / jnp.maximum(gc_ref[...], 1.0)
    g = jax.nn.silu(jnp.dot(g.astype(jnp.bfloat16), w0_ref[...],
                            preferred_element_type=jnp.float32) + b0_ref[...])
    g = jax.nn.silu(jnp.dot(g.astype(jnp.bfloat16), w1_ref[...],
                            preferred_element_type=jnp.float32) + b1_ref[...])
    o_ref[...] = (jnp.dot(g.astype(jnp.bfloat16), w2_ref[...],
                          preferred_element_type=jnp.float32) + b2_ref[...])


def _graph_mlp(gs, gc, w0, b0, w1, b1, w2, b2):
    return pl.pallas_call(
        _graph_body,
        out_shape=jax.ShapeDtypeStruct((N_GRAPHS, 1), jnp.float32),
    )(gs, gc, w0, b0, w1, b1, w2, b2)


def kernel(coords, params, atomids, edge_index, batch):
    f32 = jnp.float32
    src = edge_index[0].astype(jnp.int32)
    dst = edge_index[1].astype(jnp.int32)
    pad_e = E_PAD - N_EDGES
    src_p = jnp.concatenate([src, jnp.zeros((pad_e,), jnp.int32)])
    dst_p = jnp.concatenate([dst, jnp.zeros((pad_e,), jnp.int32)])
    idx2 = jnp.concatenate([src_p, dst_p]).reshape(2 * E_PAD // 128, 128)
    dst_sc = jnp.concatenate(
        [dst, jnp.full((pad_e,), N_NODES, jnp.int32)]).reshape(E_PAD // 128, 128)

    emb_pad = jnp.zeros((16, EMB), f32).at[:11].set(params['emb'])
    bf = jnp.bfloat16
    wi = [params['We1'][k][:EMB].astype(bf) for k in range(N_KERNELS)]
    wj = [params['We1'][k][EMB:2 * EMB].astype(bf) for k in range(N_KERNELS)]
    wea = [jnp.zeros((128, H_DIM), bf).at[:EDGE_IN - 2 * EMB].set(
        params['We1'][k][2 * EMB:].astype(bf)) for k in range(N_KERNELS)]
    be1 = [params['be1'][k].reshape(1, -1) for k in range(N_KERNELS)]
    we2 = [params['We2'][k].astype(bf) for k in range(N_KERNELS)]
    be2 = [params['be2'][k].reshape(1, -1) for k in range(N_KERNELS)]
    ln_e_g = [params['ln_e_g'][k].reshape(1, -1) for k in range(N_KERNELS)]
    ln_e_b = [params['ln_e_b'][k].reshape(1, -1) for k in range(N_KERNELS)]
    whf = [params['Wn1'][k][:EMB].astype(bf) for k in range(N_KERNELS)]
    wm = [params['Wn1'][k][EMB:].astype(bf) for k in range(N_KERNELS)]
    bn1 = [params['bn1'][k].reshape(1, -1) for k in range(N_KERNELS)]
    wn2 = [params['Wn2'][k].astype(bf) for k in range(N_KERNELS)]
    bn2 = [params['bn2'][k].reshape(1, -1) for k in range(N_KERNELS)]
    ln1g = [params['ln_n1_g'][k].reshape(1, -1) for k in range(N_KERNELS)]
    ln1b = [params['ln_n1_b'][k].reshape(1, -1) for k in range(N_KERNELS)]
    ln2g = [params['ln_n2_g'][k].reshape(1, -1) for k in range(N_KERNELS)]
    ln2b = [params['ln_n2_b'][k].reshape(1, -1) for k in range(N_KERNELS)]

    feats = _embed(atomids.astype(f32).reshape(N_NODES, 1), emb_pad)
    cpad = jnp.zeros((N_PAD, 3), f32).at[:N_NODES].set(coords)
    rd = _sc_edge_dist(cpad[:, 0], cpad[:, 1], cpad[:, 2], src_p, dst_p)
    scales = (2.0 ** jnp.arange(FF, dtype=f32)).reshape(1, FF)
    ea = _edge_attr(rd.reshape(E_PAD, 1), scales)

    flist = [feats]
    cnt = jnp.zeros((N_NODES, 1), f32)
    for k in range(N_KERNELS):
        width = 48 if k == 0 else M_DIM
        gathered = _sc_gather(feats, idx2)
        m = _edge_mlp(gathered, ea, wi[k], wj[k], wea[k], be1[k], we2[k],
                      be2[k], ln_e_g[k], ln_e_b[k], width)
        sums = _sc_scatter_add(m, dst_sc, width)
        feats, cnt_new = _node_mlp(
            sums, cnt, feats, whf[k], wm[k], bn1[k], wn2[k], bn2[k],
            ln_e_g[k], ln_e_b[k], ln1g[k], ln1b[k], ln2g[k], ln2b[k],
            first=(k == 0))
        if k == 0:
            cnt = cnt_new
        flist.append(feats)

    gs, gc = _readout(flist, params['fW0'].astype(bf),
                      params['fb0'].reshape(1, -1),
                      params['fW1'].astype(bf), params['fb1'].reshape(1, -1),
                      params['fW2'].astype(bf), params['fb2'].reshape(1, -1),
                      batch.astype(f32).reshape(N_NODES, 1))
    return _graph_mlp(gs, gc, params['gW0'].astype(bf),
                      params['gb0'].reshape(1, -1),
                      params['gW1'].astype(bf), params['gb1'].reshape(1, -1),
                      params['gW2'].astype(bf), params['gb2'].reshape(1, -1))
